# baseline (speedup 1.0000x reference)
import math

import jax
import jax.numpy as jnp
from jax import lax
from jax.experimental import pallas as pl
from jax.experimental.pallas import tpu as pltpu
from jax.experimental.pallas import tpu_sc as plsc

VOCAB = 1000000
EMBED_SIZE = 32
BATCH = 4096
HIST = 200
SCALE = math.sqrt(EMBED_SIZE)

NC = 2
NS = 16
NW = NC * NS
LANES = 16

BPW = BATCH // NW
NBUF = 5
NTR = 2


def _body(xt_hbm, lutp_hbm, out_hbm, idx_all, evt, rows, tr, gsems, osems):
    wid = lax.axis_index("s") * NC + lax.axis_index("c")
    b0 = wid * BPW

    iota16 = lax.iota(jnp.int32, LANES)

    def ev_init(e0, c):
        evt[e0, :] = (iota16 + e0) & (EMBED_SIZE - 1)
        return c

    lax.fori_loop(0, EMBED_SIZE, ev_init, 0)

    pltpu.sync_copy(xt_hbm.at[:, pl.ds(b0, BPW)], idx_all)

    def gather_h(h, b):
        pltpu.async_copy(lutp_hbm.at[idx_all.at[h]], rows[b], gsems[b])

    def gwait(h, b):
        pltpu.make_async_copy(lutp_hbm.at[idx_all.at[h]], rows[b],
                              gsems[b]).wait()

    def out_copy(h, t):
        return pltpu.make_async_copy(tr[t], out_hbm.at[h, :, pl.ds(b0, BPW)],
                                     osems[t])

    for b in range(NBUF):
        gather_h(jnp.int32(b), b)

    def step(p, carry):
        for q in range(NBUF * NTR):
            h = NBUF * NTR * p + q
            b = q % NBUF
            t = q % NTR
            gwait(h, b)

            @pl.when(h >= NTR)
            def _():
                out_copy(h - NTR, t).wait()

            def kblock(k, kc):
                ri = iota16 + 16 * k
                for e0 in range(EMBED_SIZE):
                    ev = evt[e0, :]
                    g = plsc.load_gather(rows[b], [ri, ev])
                    plsc.store_scatter(tr[t], [ev, ri], g * SCALE)
                return kc

            lax.fori_loop(0, 8, kblock, 0, unroll=2)
            out_copy(h, t).start()

            @pl.when(h + NBUF < HIST)
            def _():
                gather_h(h + NBUF, b)
        return carry

    lax.fori_loop(0, HIST // (NBUF * NTR), step, 0)

    for q in range(HIST % (NBUF * NTR)):
        h = HIST - (HIST % (NBUF * NTR)) + q
        b = q % NBUF
        t = q % NTR
        gwait(h, b)
        out_copy(h - NTR, t).wait()

        def kblock_t(k, kc):
            ri = iota16 + 16 * k
            for e0 in range(EMBED_SIZE):
                ev = evt[e0, :]
                g = plsc.load_gather(rows[b], [ri, ev])
                plsc.store_scatter(tr[t], [ev, ri], g * SCALE)
            return kc

        lax.fori_loop(0, 8, kblock_t, 0)
        out_copy(h, t).start()

    for q in range(NTR):
        out_copy(HIST - NTR + q, (HIST - NTR + q) % NTR).wait()


@jax.jit
def _lookup(x_t, lutp):
    mesh = plsc.VectorSubcoreMesh(core_axis_name="c", subcore_axis_name="s")
    return pl.kernel(
        _body,
        out_type=jax.ShapeDtypeStruct((HIST, EMBED_SIZE, BATCH), jnp.float32),
        mesh=mesh,
        scratch_types=[
            pltpu.VMEM((HIST, BPW), jnp.int32),
            pltpu.VMEM((EMBED_SIZE, LANES), jnp.int32),
            [pltpu.VMEM((BPW, 128), jnp.float32) for _ in range(NBUF)],
            [pltpu.VMEM((EMBED_SIZE, BPW), jnp.float32) for _ in range(NTR)],
            [pltpu.SemaphoreType.DMA for _ in range(NBUF)],
            [pltpu.SemaphoreType.DMA for _ in range(NTR)],
        ],
        compiler_params=pltpu.CompilerParams(
            use_tc_tiling_on_sc=True, needs_layout_passes=False),
    )(x_t, lutp)


def kernel(x, lut):
    x_t = jnp.swapaxes(x, 0, 1)
    lutp = jnp.pad(lut, ((0, 0), (0, 128 - EMBED_SIZE)))
    out = _lookup(x_t, lutp)
    return jnp.transpose(out, (2, 0, 1))


# speedup vs baseline: 1.4408x; 1.4408x over previous
import math

import jax
import jax.numpy as jnp
from jax import lax
from jax.experimental import pallas as pl
from jax.experimental.pallas import tpu as pltpu
from jax.experimental.pallas import tpu_sc as plsc

VOCAB = 1000000
EMBED_SIZE = 32
BATCH = 4096
HIST = 200
SCALE = math.sqrt(EMBED_SIZE)

NC = 2
NS = 16
NW = NC * NS
LANES = 16

BPW = BATCH // NW
NBUF = 5
NTR = 2


def _body(xt_hbm, lutp_hbm, out_hbm, idx_all, rows, tr, gsems, osems):
    wid = lax.axis_index("s") * NC + lax.axis_index("c")
    b0 = wid * BPW

    iota16 = lax.iota(jnp.int32, LANES)

    pltpu.sync_copy(xt_hbm.at[:, pl.ds(b0, BPW)], idx_all)

    def gather_h(h, b):
        pltpu.async_copy(lutp_hbm.at[idx_all.at[h]], rows[b], gsems[b])

    def gwait(h, b):
        pltpu.make_async_copy(lutp_hbm.at[idx_all.at[h]], rows[b],
                              gsems[b]).wait()

    def out_copy(h, t):
        return pltpu.make_async_copy(tr[t], out_hbm.at[h, :, pl.ds(b0, BPW)],
                                     osems[t])

    for b in range(NBUF):
        gather_h(jnp.int32(b), b)

    def step(p, carry):
        for q in range(NBUF * NTR):
            h = NBUF * NTR * p + q
            b = q % NBUF
            t = q % NTR
            gwait(h, b)

            @pl.when(h >= NTR)
            def _():
                out_copy(h - NTR, t).wait()

            def kblock(k, kc):
                ri = iota16 + 16 * k
                for e0 in range(EMBED_SIZE):
                    ev = (iota16 + e0) & (EMBED_SIZE - 1)
                    g = plsc.load_gather(rows[b], [ri, ev])
                    plsc.store_scatter(tr[t], [ev, ri], g * SCALE)
                return kc

            lax.fori_loop(0, 8, kblock, 0, unroll=2)
            out_copy(h, t).start()

            @pl.when(h + NBUF < HIST)
            def _():
                gather_h(h + NBUF, b)
        return carry

    lax.fori_loop(0, HIST // (NBUF * NTR), step, 0)

    for q in range(HIST % (NBUF * NTR)):
        h = HIST - (HIST % (NBUF * NTR)) + q
        b = q % NBUF
        t = q % NTR
        gwait(h, b)
        out_copy(h - NTR, t).wait()

        def kblock_t(k, kc):
            ri = iota16 + 16 * k
            for e0 in range(EMBED_SIZE):
                ev = (iota16 + e0) & (EMBED_SIZE - 1)
                g = plsc.load_gather(rows[b], [ri, ev])
                plsc.store_scatter(tr[t], [ev, ri], g * SCALE)
            return kc

        lax.fori_loop(0, 8, kblock_t, 0)
        out_copy(h, t).start()

    for q in range(NTR):
        out_copy(HIST - NTR + q, (HIST - NTR + q) % NTR).wait()


@jax.jit
def _lookup(x_t, lutp):
    mesh = plsc.VectorSubcoreMesh(core_axis_name="c", subcore_axis_name="s")
    return pl.kernel(
        _body,
        out_type=jax.ShapeDtypeStruct((HIST, EMBED_SIZE, BATCH), jnp.float32),
        mesh=mesh,
        scratch_types=[
            pltpu.VMEM((HIST, BPW), jnp.int32),
            [pltpu.VMEM((BPW, 128), jnp.float32) for _ in range(NBUF)],
            [pltpu.VMEM((EMBED_SIZE, BPW), jnp.float32) for _ in range(NTR)],
            [pltpu.SemaphoreType.DMA for _ in range(NBUF)],
            [pltpu.SemaphoreType.DMA for _ in range(NTR)],
        ],
        compiler_params=pltpu.CompilerParams(
            use_tc_tiling_on_sc=True, needs_layout_passes=False),
    )(x_t, lutp)


def kernel(x, lut):
    x_t = jnp.swapaxes(x, 0, 1)
    lutp = jnp.pad(lut, ((0, 0), (0, 128 - EMBED_SIZE)))
    out = _lookup(x_t, lutp)
    return jnp.transpose(out, (2, 0, 1))


# speedup vs baseline: 1.4418x; 1.0007x over previous
import math

import jax
import jax.numpy as jnp
from jax import lax
from jax.experimental import pallas as pl
from jax.experimental.pallas import tpu as pltpu
from jax.experimental.pallas import tpu_sc as plsc

VOCAB = 1000000
EMBED_SIZE = 32
BATCH = 4096
HIST = 200
SCALE = math.sqrt(EMBED_SIZE)

NC = 2
NS = 16
NW = NC * NS
LANES = 16

BPW = BATCH // NW
NBUF = 5
NTR = 2


def _body(xt_hbm, lutp_hbm, out_hbm, idx_all, rows, tr, gsems, osems):
    wid = lax.axis_index("s") * NC + lax.axis_index("c")
    b0 = wid * BPW

    iota16 = lax.iota(jnp.int32, LANES)

    pltpu.sync_copy(xt_hbm.at[:, pl.ds(b0, BPW)], idx_all)

    def gather_h(h, b):
        pltpu.async_copy(lutp_hbm.at[idx_all.at[h]], rows[b], gsems[b])

    def gwait(h, b):
        pltpu.make_async_copy(lutp_hbm.at[idx_all.at[h]], rows[b],
                              gsems[b]).wait()

    def out_copy(h, t):
        return pltpu.make_async_copy(tr[t], out_hbm.at[h, :, pl.ds(b0, BPW)],
                                     osems[t])

    for b in range(NBUF):
        gather_h(jnp.int32(b), b)

    def step(p, carry):
        for q in range(NBUF * NTR):
            h = NBUF * NTR * p + q
            b = q % NBUF
            t = q % NTR
            gwait(h, b)

            @pl.when(h >= NTR)
            def _():
                out_copy(h - NTR, t).wait()

            def kblock(k, kc):
                ri = iota16 + 16 * k
                for e0 in range(EMBED_SIZE):
                    ev = (iota16 + e0) & (EMBED_SIZE - 1)
                    g = plsc.load_gather(rows[b], [ri, ev])
                    plsc.store_scatter(tr[t], [ev, ri], g * SCALE)
                return kc

            lax.fori_loop(0, 8, kblock, 0, unroll=2)
            out_copy(h, t).start()

            @pl.when(h + NBUF < HIST)
            def _():
                gather_h(h + NBUF, b)
        return carry

    lax.fori_loop(0, HIST // (NBUF * NTR), step, 0)

    for q in range(HIST % (NBUF * NTR)):
        h = HIST - (HIST % (NBUF * NTR)) + q
        b = q % NBUF
        t = q % NTR
        gwait(h, b)
        out_copy(h - NTR, t).wait()

        def kblock_t(k, kc):
            ri = iota16 + 16 * k
            for e0 in range(EMBED_SIZE):
                ev = (iota16 + e0) & (EMBED_SIZE - 1)
                g = plsc.load_gather(rows[b], [ri, ev])
                plsc.store_scatter(tr[t], [ev, ri], g * SCALE)
            return kc

        lax.fori_loop(0, 8, kblock_t, 0)
        out_copy(h, t).start()

    for q in range(NTR):
        out_copy(HIST - NTR + q, (HIST - NTR + q) % NTR).wait()


@jax.jit
def _lookup(x_t, lutp):
    mesh = plsc.VectorSubcoreMesh(core_axis_name="c", subcore_axis_name="s")
    return pl.kernel(
        _body,
        out_type=jax.ShapeDtypeStruct((HIST, EMBED_SIZE, BATCH), jnp.float32),
        mesh=mesh,
        scratch_types=[
            pltpu.VMEM((HIST, BPW), jnp.int32),
            [pltpu.VMEM((BPW, 128), jnp.float32) for _ in range(NBUF)],
            [pltpu.VMEM((EMBED_SIZE, BPW), jnp.float32) for _ in range(NTR)],
            [pltpu.SemaphoreType.DMA for _ in range(NBUF)],
            [pltpu.SemaphoreType.DMA for _ in range(NTR)],
        ],
        compiler_params=pltpu.CompilerParams(
            use_tc_tiling_on_sc=True, needs_layout_passes=False),
    )(x_t, lutp)


def kernel(x, lut):
    x_t = jnp.swapaxes(x.astype(jnp.int32), 0, 1)
    lutp = jnp.pad(lut, ((0, 0), (0, 128 - EMBED_SIZE)))
    out = _lookup(x_t, lutp)
    return jnp.transpose(out, (2, 0, 1))
